# Initial kernel scaffold; baseline (speedup 1.0000x reference)
import functools

import jax
import jax.numpy as jnp
from jax import lax
from jax.experimental import pallas as pl
from jax.experimental.pallas import tpu as pltpu
from jax.experimental.pallas import tpu_sc as plsc

_NGAUSS = 50
_CUTOFF = 6.0
_NGRAPH = 8
_NI = 4

_BE = 2000
_BN = 2000
_EBLK = 128
_NSUB = 16
_NCORE = 2


def _swish(v):
    return v * jax.nn.sigmoid(v)


def _mm_t(a, w):
    return lax.dot_general(a, w, (((1,), (1,)), ((), ())),
                           preferred_element_type=jnp.float32)


def _edge_body(rel_ref, e1w, e1b, e2w, e2b, e3w, e3b, gw, gb, out_ref):
    rp3 = rel_ref[...]
    dist = jnp.sqrt(jnp.sum(rp3 * rp3, axis=-1, keepdims=True) + 1e-12)
    step = _CUTOFF / (_NGAUSS - 1)
    offset = step * lax.broadcasted_iota(jnp.float32, (1, _NGAUSS), 1)
    coeff = -0.5 / step ** 2
    ea = jnp.exp(coeff * (dist - offset) ** 2)
    rp = _mm_t(rp3, e1w[...]) + e1b[...]
    ea = _mm_t(ea, e2w[...]) + e2b[...]
    e = _swish(jnp.concatenate([rp, ea], axis=1))
    e = _swish(_mm_t(e, e3w[...]) + e3b[...])
    ei = _swish(_mm_t(e, gw[...]) + gb[...])
    out_ref[0] = ei[:, :32]
    out_ref[1] = ei[:, 32:]


def _edge_filter(rel_pos, e1w, e1b, e2w, e2b, e3w, e3b, gw, gb):
    E = rel_pos.shape[0]
    grid = E // _BE
    full = lambda a: pl.BlockSpec(a.shape, lambda b: (0,) * a.ndim)
    return pl.pallas_call(
        _edge_body,
        grid=(grid,),
        in_specs=[
            pl.BlockSpec((_BE, 3), lambda b: (b, 0)),
            full(e1w), full(e1b), full(e2w), full(e2b),
            full(e3w), full(e3b), full(gw), full(gb),
        ],
        out_specs=pl.BlockSpec((2, _BE, 32), lambda b: (0, b, 0)),
        out_shape=jax.ShapeDtypeStruct((2, E, 32), jnp.float32),
    )(rel_pos, e1w, e1b, e2w, e2b, e3w, e3b, gw, gb)


def _node_body(x_ref, new, neb, lw, lb, l2w, l2b, dw, db, h_ref, hd_ref):
    h0 = _mm_t(x_ref[...], new[...]) + neb[...]
    h = _swish(_mm_t(h0, lw[...]) + lb[...])
    h = _swish(_mm_t(h, l2w[...]) + l2b[...])
    h_ref[...] = h
    hd = _swish(_mm_t(h, dw[...]) + db[...])
    hd_ref[0] = hd[:, :32]
    hd_ref[1] = hd[:, 32:]


def _node_embed(x, new, neb, lw, lb, l2w, l2b, dw, db):
    N = x.shape[0]
    grid = N // _BN
    full = lambda a: pl.BlockSpec(a.shape, lambda b: (0,) * a.ndim)
    return pl.pallas_call(
        _node_body,
        grid=(grid,),
        in_specs=[
            pl.BlockSpec((_BN, x.shape[1]), lambda b: (b, 0)),
            full(new), full(neb), full(lw), full(lb),
            full(l2w), full(l2b), full(dw), full(db),
        ],
        out_specs=[
            pl.BlockSpec((_BN, 64), lambda b: (b, 0)),
            pl.BlockSpec((2, _BN, 32), lambda b: (0, b, 0)),
        ],
        out_shape=[
            jax.ShapeDtypeStruct((N, 64), jnp.float32),
            jax.ShapeDtypeStruct((2, N, 32), jnp.float32),
        ],
    )(x, new, neb, lw, lb, l2w, l2b, dw, db)


def _sc_gather_scatter(ei, edge_index, hd_split):
    N = hd_split.shape[1]
    E = edge_index.shape[1]
    nblk = E // _EBLK
    stripe = N // _NSUB

    mesh = plsc.VectorSubcoreMesh(core_axis_name="c", subcore_axis_name="s")

    @functools.partial(
        pl.kernel,
        out_type=jax.ShapeDtypeStruct((2, N, 32), jnp.float32),
        mesh=mesh,
        scratch_types=[
            pltpu.VMEM((_EBLK,), jnp.int32),
            pltpu.VMEM((_EBLK,), jnp.int32),
            pltpu.VMEM((_EBLK, 32), jnp.float32),
            pltpu.VMEM((_EBLK, 32), jnp.float32),
            pltpu.VMEM((125, 32), jnp.float32),
            pltpu.VMEM_SHARED((N, 32), jnp.float32),
        ],
    )
    def k(ei_hbm, edge_hbm, hd_hbm, out_hbm, src_v, dst_v, rows_v, eiv,
          zero_v, acc):
        c = lax.axis_index("c")
        s = lax.axis_index("s")

        z16 = jnp.zeros((16,), jnp.float32)

        @pl.loop(0, 125)
        def _(r):
            zero_v[r, pl.ds(0, 16)] = z16
            zero_v[r, pl.ds(16, 16)] = z16

        @pl.loop(0, stripe // 125)
        def _(t):
            pltpu.sync_copy(zero_v, acc.at[pl.ds(s * stripe + t * 125, 125)])

        plsc.subcore_barrier()

        @pl.loop(s, nblk, step=_NSUB)
        def _(b):
            base = b * _EBLK
            pltpu.sync_copy(edge_hbm.at[0, pl.ds(base, _EBLK)], src_v)
            pltpu.sync_copy(edge_hbm.at[1, pl.ds(base, _EBLK)], dst_v)
            pltpu.sync_copy(ei_hbm.at[c, pl.ds(base, _EBLK)], eiv)
            pltpu.sync_copy(hd_hbm.at[c].at[src_v], rows_v)

            @pl.loop(0, _EBLK)
            def _(r):
                rows_v[r, pl.ds(0, 16)] = rows_v[r, pl.ds(0, 16)] * eiv[r, pl.ds(0, 16)]
                rows_v[r, pl.ds(16, 16)] = rows_v[r, pl.ds(16, 16)] * eiv[r, pl.ds(16, 16)]

            pltpu.sync_copy(rows_v, acc.at[dst_v], add=True)

        plsc.subcore_barrier()

        pltpu.sync_copy(acc.at[pl.ds(s * stripe, stripe)],
                        out_hbm.at[c, pl.ds(s * stripe, stripe)])

    return k(ei, edge_index, hd_split)


def _stats_body(agg_ref, out_ref, acc_ref):
    i = pl.program_id(0)
    a = jnp.concatenate([agg_ref[0], agg_ref[1]], axis=-1)

    @pl.when(i == 0)
    def _():
        acc_ref[...] = jnp.zeros_like(acc_ref)

    acc_ref[0:1] = acc_ref[0:1] + jnp.sum(a, axis=0, keepdims=True)
    acc_ref[1:2] = acc_ref[1:2] + jnp.sum(a * a, axis=0, keepdims=True)

    @pl.when(i == pl.num_programs(0) - 1)
    def _():
        out_ref[...] = acc_ref[...]


def _stats(agg):
    N = agg.shape[1]
    grid = N // _BN
    return pl.pallas_call(
        _stats_body,
        grid=(grid,),
        in_specs=[pl.BlockSpec((2, _BN, 32), lambda b: (0, b, 0))],
        out_specs=pl.BlockSpec((2, 64), lambda b: (0, 0)),
        out_shape=jax.ShapeDtypeStruct((2, 64), jnp.float32),
        scratch_shapes=[pltpu.VMEM((2, 64), jnp.float32)],
    )(agg)


def _update_body(n_nodes, has_down, agg_ref, h_ref, st_ref, gn_g, gn_b, gn_a,
                 uw, ub, dw, db, h_out, hd_out):
    a = jnp.concatenate([agg_ref[0], agg_ref[1]], axis=-1)
    m1 = st_ref[0:1] / n_nodes
    m2 = st_ref[1:2] / n_nodes
    alpha = gn_a[...][None, :]
    var = m2 - (2.0 * alpha - alpha * alpha) * m1 * m1
    cen = a - alpha * m1
    hn = gn_g[...][None, :] * cen / jnp.sqrt(var + 1e-5) + gn_b[...][None, :]
    hn = _swish(hn)
    hn = _swish(_mm_t(hn, uw[...]) + ub[...])
    h_new = h_ref[...] + hn
    h_out[...] = h_new
    if has_down:
        hd = _swish(_mm_t(h_new, dw[...]) + db[...])
        hd_out[0] = hd[:, :32]
        hd_out[1] = hd[:, 32:]
    else:
        hd_out[...] = jnp.zeros_like(hd_out)


def _update(agg, h, st, gn_g, gn_b, gn_a, uw, ub, dw, db):
    N = h.shape[0]
    grid = N // _BN
    has_down = dw is not None
    full = lambda a: pl.BlockSpec(a.shape, lambda b: (0,) * a.ndim)
    if not has_down:
        dw = jnp.zeros((64, 64), jnp.float32)
        db = jnp.zeros((64,), jnp.float32)
    out_specs = [pl.BlockSpec((_BN, 64), lambda b: (b, 0))]
    out_shape = [jax.ShapeDtypeStruct((N, 64), jnp.float32)]
    if has_down:
        out_specs.append(pl.BlockSpec((2, _BN, 32), lambda b: (0, b, 0)))
        out_shape.append(jax.ShapeDtypeStruct((2, N, 32), jnp.float32))
    else:
        out_specs.append(pl.BlockSpec((8, 128), lambda b: (0, 0)))
        out_shape.append(jax.ShapeDtypeStruct((8, 128), jnp.float32))
    res = pl.pallas_call(
        functools.partial(_update_body, float(N), has_down),
        grid=(grid,),
        in_specs=[
            pl.BlockSpec((2, _BN, 32), lambda b: (0, b, 0)),
            pl.BlockSpec((_BN, 64), lambda b: (b, 0)),
            full(st), full(gn_g), full(gn_b), full(gn_a),
            full(uw), full(ub), full(dw), full(db),
        ],
        out_specs=out_specs,
        out_shape=out_shape,
    )(agg, h, st, gn_g, gn_b, gn_a, uw, ub, dw, db)
    if has_down:
        return res[0], res[1]
    return res[0], None


def _head_body(h_ref, batch_ref, o1w, o1b, o2w, o2b, out_ref, acc_ref):
    i = pl.program_id(0)
    ho = _swish(_mm_t(h_ref[...], o1w[...]) + o1b[...])
    ho2 = _mm_t(ho, o2w[...]) + o2b[...]
    b = jnp.reshape(batch_ref[0], (-1, 1))
    onehot = b == lax.broadcasted_iota(jnp.int32, (1, _NGRAPH), 1)
    part = jnp.sum(jnp.where(onehot, ho2, 0.0), axis=0)

    @pl.when(i == 0)
    def _():
        acc_ref[...] = jnp.zeros_like(acc_ref)

    acc_ref[...] = acc_ref[...] + part[None, :]

    @pl.when(i == pl.num_programs(0) - 1)
    def _():
        out_ref[...] = acc_ref[...]


def _head(h, batch3, o1w, o1b, o2w, o2b):
    N = h.shape[0]
    grid = N // _BN
    full = lambda a: pl.BlockSpec(a.shape, lambda b: (0,) * a.ndim)
    return pl.pallas_call(
        _head_body,
        grid=(grid,),
        in_specs=[
            pl.BlockSpec((_BN, 64), lambda b: (b, 0)),
            pl.BlockSpec((1, 1, _BN), lambda b: (b, 0, 0)),
            full(o1w), full(o1b), full(o2w), full(o2b),
        ],
        out_specs=pl.BlockSpec((1, _NGRAPH), lambda b: (0, 0)),
        out_shape=jax.ShapeDtypeStruct((1, _NGRAPH), jnp.float32),
        scratch_shapes=[pltpu.VMEM((1, _NGRAPH), jnp.float32)],
    )(h, batch3, o1w, o1b, o2w, o2b)


def kernel(x, rel_pos, edge_index, batch, node_emb_w, node_emb_b, lin_w,
           lin_b, lin2_w, lin2_b, e1_w, e1_b, e2_w, e2_b, e3_w, e3_b, geom_w,
           geom_b, down_w, down_b, up_w, up_b, gn_gamma, gn_beta, gn_alpha,
           out1_w, out1_b, out2_w, out2_b):
    N = x.shape[0]

    ei_list = [
        _edge_filter(rel_pos, e1_w, e1_b, e2_w, e2_b, e3_w, e3_b,
                     geom_w[i], geom_b[i])
        for i in range(_NI)
    ]

    h, hd = _node_embed(x, node_emb_w, node_emb_b, lin_w, lin_b, lin2_w,
                        lin2_b, down_w[0], down_b[0])

    for i in range(_NI):
        agg = _sc_gather_scatter(ei_list[i], edge_index, hd)
        st = _stats(agg)
        if i + 1 < _NI:
            h, hd = _update(agg, h, st, gn_gamma[i], gn_beta[i], gn_alpha[i],
                            up_w[i], up_b[i], down_w[i + 1], down_b[i + 1])
        else:
            h, _ = _update(agg, h, st, gn_gamma[i], gn_beta[i], gn_alpha[i],
                           up_w[i], up_b[i], None, None)

    batch3 = batch.reshape(N // _BN, 1, _BN)
    out = _head(h, batch3, out1_w, out1_b, out2_w, out2_b)
    return out.reshape(_NGRAPH, 1)


# speedup vs baseline: 1.8463x; 1.8463x over previous
import functools

import jax
import jax.numpy as jnp
from jax import lax
from jax.experimental import pallas as pl
from jax.experimental.pallas import tpu as pltpu
from jax.experimental.pallas import tpu_sc as plsc

_NGAUSS = 50
_CUTOFF = 6.0
_NGRAPH = 8
_NI = 4

_BE = 2000
_BN = 2000
_EBLK = 128
_NSUB = 16
_NCORE = 2


def _swish(v):
    return v * jax.nn.sigmoid(v)


def _mm_t(a, w):
    return lax.dot_general(a, w, (((1,), (1,)), ((), ())),
                           preferred_element_type=jnp.float32)


def _edge_body(rel_ref, e1w, e1b, e2w, e2b, e3w, e3b, gw, gb, out_ref):
    rp3 = rel_ref[...]
    dist = jnp.sqrt(jnp.sum(rp3 * rp3, axis=-1, keepdims=True) + 1e-12)
    step = _CUTOFF / (_NGAUSS - 1)
    offset = step * lax.broadcasted_iota(jnp.int32, (1, _NGAUSS), 1).astype(jnp.float32)
    coeff = -0.5 / step ** 2
    ea = jnp.exp(coeff * (dist - offset) ** 2)
    rp = _mm_t(rp3, e1w[...]) + e1b[...]
    ea = _mm_t(ea, e2w[...]) + e2b[...]
    e = _swish(jnp.concatenate([rp, ea], axis=1))
    e = _swish(_mm_t(e, e3w[...]) + e3b[...])
    ei = _swish(_mm_t(e, gw[...]) + gb[...])
    out_ref[0] = ei[:, :32]
    out_ref[1] = ei[:, 32:]


def _edge_filter(rel_pos, e1w, e1b, e2w, e2b, e3w, e3b, gw, gb):
    E = rel_pos.shape[0]
    grid = E // _BE
    full = lambda a: pl.BlockSpec(a.shape, lambda b: (0,) * a.ndim)
    return pl.pallas_call(
        _edge_body,
        grid=(grid,),
        in_specs=[
            pl.BlockSpec((_BE, 3), lambda b: (b, 0)),
            full(e1w), full(e1b), full(e2w), full(e2b),
            full(e3w), full(e3b), full(gw), full(gb),
        ],
        out_specs=pl.BlockSpec((2, _BE, 32), lambda b: (0, b, 0)),
        out_shape=jax.ShapeDtypeStruct((2, E, 32), jnp.float32),
    )(rel_pos, e1w, e1b, e2w, e2b, e3w, e3b, gw, gb)


def _node_body(x_ref, new, neb, lw, lb, l2w, l2b, dw, db, h_ref, hd_ref):
    h0 = _mm_t(x_ref[...], new[...]) + neb[...]
    h = _swish(_mm_t(h0, lw[...]) + lb[...])
    h = _swish(_mm_t(h, l2w[...]) + l2b[...])
    h_ref[...] = h
    hd = _swish(_mm_t(h, dw[...]) + db[...])
    hd_ref[0] = hd[:, :32]
    hd_ref[1] = hd[:, 32:]


def _node_embed(x, new, neb, lw, lb, l2w, l2b, dw, db):
    N = x.shape[0]
    grid = N // _BN
    full = lambda a: pl.BlockSpec(a.shape, lambda b: (0,) * a.ndim)
    return pl.pallas_call(
        _node_body,
        grid=(grid,),
        in_specs=[
            pl.BlockSpec((_BN, x.shape[1]), lambda b: (b, 0)),
            full(new), full(neb), full(lw), full(lb),
            full(l2w), full(l2b), full(dw), full(db),
        ],
        out_specs=[
            pl.BlockSpec((_BN, 64), lambda b: (b, 0)),
            pl.BlockSpec((2, _BN, 32), lambda b: (0, b, 0)),
        ],
        out_shape=[
            jax.ShapeDtypeStruct((N, 64), jnp.float32),
            jax.ShapeDtypeStruct((2, N, 32), jnp.float32),
        ],
    )(x, new, neb, lw, lb, l2w, l2b, dw, db)


_ZBLK = 200


def _sc_gather_scatter(ei, src, dst, hd_split):
    N = hd_split.shape[1]
    E = src.shape[0]
    nblk = E // _EBLK
    nzblk = N // _ZBLK

    mesh = plsc.VectorSubcoreMesh(core_axis_name="c", subcore_axis_name="s")

    @functools.partial(
        pl.kernel,
        out_type=jax.ShapeDtypeStruct((2, N, 32), jnp.float32),
        mesh=mesh,
        scratch_types=[
            pltpu.VMEM((_EBLK,), jnp.int32),
            pltpu.VMEM((_EBLK,), jnp.int32),
            pltpu.VMEM((_EBLK, 32), jnp.float32),
            pltpu.VMEM((_EBLK, 32), jnp.float32),
            pltpu.VMEM((_ZBLK, 32), jnp.float32),
            pltpu.VMEM_SHARED((N, 32), jnp.float32),
        ],
        compiler_params=pltpu.CompilerParams(use_tc_tiling_on_sc=False),
    )
    def k(ei_hbm, src_hbm, dst_hbm, hd_hbm, out_hbm, src_v, dst_v, rows_v,
          eiv, zero_v, acc):
        c = lax.axis_index("c")
        s = lax.axis_index("s")

        z16 = jnp.zeros((16,), jnp.float32)

        @pl.loop(0, _ZBLK)
        def _(r):
            zero_v[r, pl.ds(0, 16)] = z16
            zero_v[r, pl.ds(16, 16)] = z16

        @pl.loop(s, nzblk, step=_NSUB)
        def _(t):
            pltpu.sync_copy(zero_v, acc.at[pl.ds(t * _ZBLK, _ZBLK)])

        plsc.subcore_barrier()

        @pl.loop(s, nblk, step=_NSUB)
        def _(b):
            base = b * _EBLK
            pltpu.sync_copy(src_hbm.at[pl.ds(base, _EBLK)], src_v)
            pltpu.sync_copy(dst_hbm.at[pl.ds(base, _EBLK)], dst_v)
            pltpu.sync_copy(ei_hbm.at[c, pl.ds(base, _EBLK)], eiv)
            pltpu.sync_copy(hd_hbm.at[c].at[src_v], rows_v)

            @pl.loop(0, _EBLK)
            def _(r):
                rows_v[r, pl.ds(0, 16)] = rows_v[r, pl.ds(0, 16)] * eiv[r, pl.ds(0, 16)]
                rows_v[r, pl.ds(16, 16)] = rows_v[r, pl.ds(16, 16)] * eiv[r, pl.ds(16, 16)]

            pltpu.sync_copy(rows_v, acc.at[dst_v], add=True)

        plsc.subcore_barrier()

        @pl.loop(s, nzblk, step=_NSUB)
        def _(t):
            pltpu.sync_copy(acc.at[pl.ds(t * _ZBLK, _ZBLK)],
                            out_hbm.at[c, pl.ds(t * _ZBLK, _ZBLK)])

    return k(ei, src, dst, hd_split)


def _stats_body(agg_ref, out_ref, acc_ref):
    i = pl.program_id(0)
    a = jnp.concatenate([agg_ref[0], agg_ref[1]], axis=-1)

    @pl.when(i == 0)
    def _():
        acc_ref[...] = jnp.zeros_like(acc_ref)

    acc_ref[0:1] = acc_ref[0:1] + jnp.sum(a, axis=0, keepdims=True)
    acc_ref[1:2] = acc_ref[1:2] + jnp.sum(a * a, axis=0, keepdims=True)

    @pl.when(i == pl.num_programs(0) - 1)
    def _():
        out_ref[...] = acc_ref[...]


def _stats(agg):
    N = agg.shape[1]
    grid = N // _BN
    return pl.pallas_call(
        _stats_body,
        grid=(grid,),
        in_specs=[pl.BlockSpec((2, _BN, 32), lambda b: (0, b, 0))],
        out_specs=pl.BlockSpec((2, 64), lambda b: (0, 0)),
        out_shape=jax.ShapeDtypeStruct((2, 64), jnp.float32),
        scratch_shapes=[pltpu.VMEM((2, 64), jnp.float32)],
    )(agg)


def _update_body(n_nodes, has_down, agg_ref, h_ref, st_ref, gn_g, gn_b, gn_a,
                 uw, ub, dw, db, h_out, hd_out):
    a = jnp.concatenate([agg_ref[0], agg_ref[1]], axis=-1)
    m1 = st_ref[0:1] / n_nodes
    m2 = st_ref[1:2] / n_nodes
    alpha = gn_a[...][None, :]
    var = m2 - (2.0 * alpha - alpha * alpha) * m1 * m1
    cen = a - alpha * m1
    hn = gn_g[...][None, :] * cen / jnp.sqrt(var + 1e-5) + gn_b[...][None, :]
    hn = _swish(hn)
    hn = _swish(_mm_t(hn, uw[...]) + ub[...])
    h_new = h_ref[...] + hn
    h_out[...] = h_new
    if has_down:
        hd = _swish(_mm_t(h_new, dw[...]) + db[...])
        hd_out[0] = hd[:, :32]
        hd_out[1] = hd[:, 32:]
    else:
        hd_out[...] = jnp.zeros_like(hd_out)


def _update(agg, h, st, gn_g, gn_b, gn_a, uw, ub, dw, db):
    N = h.shape[0]
    grid = N // _BN
    has_down = dw is not None
    full = lambda a: pl.BlockSpec(a.shape, lambda b: (0,) * a.ndim)
    if not has_down:
        dw = jnp.zeros((64, 64), jnp.float32)
        db = jnp.zeros((64,), jnp.float32)
    out_specs = [pl.BlockSpec((_BN, 64), lambda b: (b, 0))]
    out_shape = [jax.ShapeDtypeStruct((N, 64), jnp.float32)]
    if has_down:
        out_specs.append(pl.BlockSpec((2, _BN, 32), lambda b: (0, b, 0)))
        out_shape.append(jax.ShapeDtypeStruct((2, N, 32), jnp.float32))
    else:
        out_specs.append(pl.BlockSpec((8, 128), lambda b: (0, 0)))
        out_shape.append(jax.ShapeDtypeStruct((8, 128), jnp.float32))
    res = pl.pallas_call(
        functools.partial(_update_body, float(N), has_down),
        grid=(grid,),
        in_specs=[
            pl.BlockSpec((2, _BN, 32), lambda b: (0, b, 0)),
            pl.BlockSpec((_BN, 64), lambda b: (b, 0)),
            full(st), full(gn_g), full(gn_b), full(gn_a),
            full(uw), full(ub), full(dw), full(db),
        ],
        out_specs=out_specs,
        out_shape=out_shape,
    )(agg, h, st, gn_g, gn_b, gn_a, uw, ub, dw, db)
    if has_down:
        return res[0], res[1]
    return res[0], None


def _head_body(h_ref, batch_ref, o1w, o1b, o2w, o2b, out_ref, acc_ref):
    i = pl.program_id(0)
    ho = _swish(_mm_t(h_ref[...], o1w[...]) + o1b[...])
    b2d = batch_ref[0]
    onehot_t = (lax.broadcasted_iota(jnp.int32, (_NGRAPH, b2d.shape[1]), 0)
                == b2d).astype(jnp.float32)
    m = lax.dot_general(onehot_t, ho, (((1,), (0,)), ((), ())),
                        preferred_element_type=jnp.float32)
    part = jnp.sum(m * o2w[...], axis=1)
    cnt = jnp.sum(onehot_t, axis=1)

    @pl.when(i == 0)
    def _():
        acc_ref[...] = jnp.zeros_like(acc_ref)

    acc_ref[0:1] = acc_ref[0:1] + part[None, :]
    acc_ref[1:2] = acc_ref[1:2] + cnt[None, :]

    @pl.when(i == pl.num_programs(0) - 1)
    def _():
        out_ref[...] = acc_ref[0:1] + o2b[0] * acc_ref[1:2]


def _head(h, batch3, o1w, o1b, o2w, o2b):
    N = h.shape[0]
    grid = N // _BN
    full = lambda a: pl.BlockSpec(a.shape, lambda b: (0,) * a.ndim)
    return pl.pallas_call(
        _head_body,
        grid=(grid,),
        in_specs=[
            pl.BlockSpec((_BN, 64), lambda b: (b, 0)),
            pl.BlockSpec((1, 1, _BN), lambda b: (b, 0, 0)),
            full(o1w), full(o1b), full(o2w),
            pl.BlockSpec(memory_space=pltpu.SMEM),
        ],
        out_specs=pl.BlockSpec((1, _NGRAPH), lambda b: (0, 0)),
        out_shape=jax.ShapeDtypeStruct((1, _NGRAPH), jnp.float32),
        scratch_shapes=[pltpu.VMEM((2, _NGRAPH), jnp.float32)],
    )(h, batch3, o1w, o1b, o2w, o2b)


def kernel(x, rel_pos, edge_index, batch, node_emb_w, node_emb_b, lin_w,
           lin_b, lin2_w, lin2_b, e1_w, e1_b, e2_w, e2_b, e3_w, e3_b, geom_w,
           geom_b, down_w, down_b, up_w, up_b, gn_gamma, gn_beta, gn_alpha,
           out1_w, out1_b, out2_w, out2_b):
    N = x.shape[0]

    ei_list = [
        _edge_filter(rel_pos, e1_w, e1_b, e2_w, e2_b, e3_w, e3_b,
                     geom_w[i], geom_b[i])
        for i in range(_NI)
    ]

    h, hd = _node_embed(x, node_emb_w, node_emb_b, lin_w, lin_b, lin2_w,
                        lin2_b, down_w[0], down_b[0])

    src = edge_index[0]
    dst = edge_index[1]

    for i in range(_NI):
        agg = _sc_gather_scatter(ei_list[i], src, dst, hd)
        st = _stats(agg)
        if i + 1 < _NI:
            h, hd = _update(agg, h, st, gn_gamma[i], gn_beta[i], gn_alpha[i],
                            up_w[i], up_b[i], down_w[i + 1], down_b[i + 1])
        else:
            h, _ = _update(agg, h, st, gn_gamma[i], gn_beta[i], gn_alpha[i],
                           up_w[i], up_b[i], None, None)

    batch3 = batch.reshape(N // _BN, 1, _BN)
    out = _head(h, batch3, out1_w, out1_b, out2_w, out2_b)
    return out.reshape(_NGRAPH, 1)
